# Initial kernel scaffold; baseline (speedup 1.0000x reference)
import jax
import jax.numpy as jnp
from jax.experimental import pallas as pl


def kernel(h, edge_index, W, b):
    raise NotImplementedError("write your pallas kernel here")


# speedup vs baseline: 31.6112x; 31.6112x over previous
import functools

import jax
import jax.numpy as jnp
from jax import lax
from jax.experimental import pallas as pl
from jax.experimental.pallas import tpu as pltpu
from jax.experimental.pallas import tpu_sc as plsc

_N_NODES = 10000
_N_EDGES = 320000
_D_FEAT = 128

_NC = 2
_NS = 16
_NW = _NC * _NS
_EPT = _N_EDGES // _NW
_L = 16


def _tc_body(h_ref, w2_ref, b_ref, pq_ref):
    pq = lax.dot_general(
        w2_ref[...], h_ref[...],
        dimension_numbers=(((1,), (1,)), ((), ())),
        preferred_element_type=jnp.float32,
    )
    pq_ref[...] = pq + 0.5 * b_ref[0]


def _make_sc_kernel():
    mesh = plsc.VectorSubcoreMesh(core_axis_name="c", subcore_axis_name="s")

    @functools.partial(
        pl.kernel,
        mesh=mesh,
        out_type=jax.ShapeDtypeStruct((_N_EDGES,), jnp.float32),
        compiler_params=pltpu.CompilerParams(needs_layout_passes=False),
        scratch_types=[
            pltpu.VMEM((_N_NODES,), jnp.float32),
            pltpu.VMEM((_N_NODES,), jnp.float32),
            pltpu.VMEM((_EPT,), jnp.int32),
            pltpu.VMEM((_EPT,), jnp.int32),
            pltpu.VMEM((_EPT,), jnp.float32),
        ],
    )
    def sc_k(pq_hbm, ei_hbm, out_hbm, p_v, q_v, src_v, dst_v, o_v):
        wid = lax.axis_index("s") * _NC + lax.axis_index("c")
        base = wid * _EPT
        pltpu.sync_copy(pq_hbm.at[pl.ds(0, _N_NODES)], p_v)
        pltpu.sync_copy(pq_hbm.at[pl.ds(_N_NODES, _N_NODES)], q_v)
        pltpu.sync_copy(ei_hbm.at[pl.ds(base, _EPT)], src_v)
        pltpu.sync_copy(ei_hbm.at[pl.ds(_N_EDGES + base, _EPT)], dst_v)

        def body(i, carry):
            off = i * _L
            sidx = src_v[pl.ds(off, _L)]
            didx = dst_v[pl.ds(off, _L)]
            pv = plsc.load_gather(p_v, [sidx])
            qv = plsc.load_gather(q_v, [didx])
            x = pv + qv
            o_v[pl.ds(off, _L)] = 1.0 / (1.0 + jnp.exp(-x))
            return carry

        lax.fori_loop(0, _EPT // _L, body, 0)
        pltpu.sync_copy(o_v, out_hbm.at[pl.ds(base, _EPT)])

    return sc_k


_sc_kernel = _make_sc_kernel()


def kernel(h, edge_index, W, b):
    w2 = W.reshape(2, _D_FEAT)
    ei = edge_index.astype(jnp.int32).reshape(-1)
    pq = pl.pallas_call(
        _tc_body,
        out_shape=jax.ShapeDtypeStruct((2, _N_NODES), jnp.float32),
        in_specs=[
            pl.BlockSpec(memory_space=pltpu.VMEM),
            pl.BlockSpec(memory_space=pltpu.VMEM),
            pl.BlockSpec(memory_space=pltpu.SMEM),
        ],
        out_specs=pl.BlockSpec(memory_space=pltpu.VMEM),
    )(h, w2, b)
    scores = _sc_kernel(pq.reshape(-1), ei)
    out = scores.reshape(_N_EDGES, 1)
    return (out, out)


# speedup vs baseline: 42.1286x; 1.3327x over previous
import functools

import jax
import jax.numpy as jnp
from jax import lax
from jax.experimental import pallas as pl
from jax.experimental.pallas import tpu as pltpu
from jax.experimental.pallas import tpu_sc as plsc

_N_NODES = 10000
_N_EDGES = 320000
_D_FEAT = 128

_NC = 2
_NS = 16
_NW = _NC * _NS
_EPT = _N_EDGES // _NW
_L = 16


def _tc_body(h_ref, w2_ref, b_ref, pq_ref):
    pq = lax.dot_general(
        w2_ref[...], h_ref[...],
        dimension_numbers=(((1,), (1,)), ((), ())),
        preferred_element_type=jnp.float32,
    )
    bias = 0.5 * b_ref[0]
    pq_ref[pl.ds(0, _N_NODES)] = pq[0, :] + bias
    pq_ref[pl.ds(_N_NODES, _N_NODES)] = pq[1, :] + bias


def _make_sc_kernel():
    mesh = plsc.VectorSubcoreMesh(core_axis_name="c", subcore_axis_name="s")

    @functools.partial(
        pl.kernel,
        mesh=mesh,
        out_type=jax.ShapeDtypeStruct((_N_EDGES,), jnp.float32),
        compiler_params=pltpu.CompilerParams(needs_layout_passes=False),
        scratch_types=[
            pltpu.VMEM((_N_NODES,), jnp.float32),
            pltpu.VMEM((_N_NODES,), jnp.float32),
            pltpu.VMEM((_EPT,), jnp.int32),
            pltpu.VMEM((_EPT,), jnp.int32),
            pltpu.VMEM((_EPT,), jnp.float32),
            pltpu.SemaphoreType.DMA,
        ],
    )
    def sc_k(pq_hbm, ei_hbm, out_hbm, p_v, q_v, src_v, dst_v, o_v, sem):
        wid = lax.axis_index("s") * _NC + lax.axis_index("c")
        base = wid * _EPT
        c1 = pltpu.async_copy(pq_hbm.at[pl.ds(0, _N_NODES)], p_v, sem)
        c2 = pltpu.async_copy(pq_hbm.at[pl.ds(_N_NODES, _N_NODES)], q_v, sem)
        c3 = pltpu.async_copy(ei_hbm.at[pl.ds(base, _EPT)], src_v, sem)
        c4 = pltpu.async_copy(
            ei_hbm.at[pl.ds(_N_EDGES + base, _EPT)], dst_v, sem)
        c1.wait()
        c2.wait()
        c3.wait()
        c4.wait()

        @plsc.parallel_loop(0, _EPT, step=_L, unroll=8)
        def _(off):
            sidx = src_v[pl.ds(off, _L)]
            didx = dst_v[pl.ds(off, _L)]
            pv = plsc.load_gather(p_v, [sidx])
            qv = plsc.load_gather(q_v, [didx])
            x = pv + qv
            o_v[pl.ds(off, _L)] = 1.0 / (1.0 + jnp.exp(-x))

        pltpu.sync_copy(o_v, out_hbm.at[pl.ds(base, _EPT)])

    return sc_k


_sc_kernel = _make_sc_kernel()


def kernel(h, edge_index, W, b):
    w2 = W.reshape(2, _D_FEAT)
    ei = edge_index.astype(jnp.int32).reshape(-1)
    pq = pl.pallas_call(
        _tc_body,
        out_shape=jax.ShapeDtypeStruct((2 * _N_NODES,), jnp.float32),
        in_specs=[
            pl.BlockSpec(memory_space=pltpu.VMEM),
            pl.BlockSpec(memory_space=pltpu.VMEM),
            pl.BlockSpec(memory_space=pltpu.SMEM),
        ],
        out_specs=pl.BlockSpec(memory_space=pltpu.VMEM),
    )(h, w2, b)
    scores = _sc_kernel(pq, ei)
    out = scores.reshape(_N_EDGES, 1)
    return (out, out)


# speedup vs baseline: 48.1462x; 1.1428x over previous
import functools

import jax
import jax.numpy as jnp
from jax import lax
from jax.experimental import pallas as pl
from jax.experimental.pallas import tpu as pltpu
from jax.experimental.pallas import tpu_sc as plsc

_N_NODES = 10000
_N_EDGES = 320000
_D_FEAT = 128

_NC = 2
_NS = 16
_NW = _NC * _NS
_EPT = _N_EDGES // _NW
_L = 16
_ALN = 128
_SZ = -(-_EPT // _ALN) * _ALN


def _tc_body(h_ref, w2_ref, b_ref, pq_ref):
    pq = lax.dot_general(
        w2_ref[...], h_ref[...],
        dimension_numbers=(((1,), (1,)), ((), ())),
        preferred_element_type=jnp.float32,
    )
    bias = 0.5 * b_ref[0]
    pq_ref[pl.ds(0, _N_NODES)] = pq[0, :] + bias
    pq_ref[pl.ds(_N_NODES, _N_NODES)] = pq[1, :] + bias


def _make_sc_kernel():
    mesh = plsc.VectorSubcoreMesh(core_axis_name="c", subcore_axis_name="s")

    @functools.partial(
        pl.kernel,
        mesh=mesh,
        out_type=jax.ShapeDtypeStruct((_N_EDGES,), jnp.float32),
        compiler_params=pltpu.CompilerParams(needs_layout_passes=False),
        scratch_types=[
            pltpu.VMEM((_N_NODES,), jnp.float32),
            pltpu.VMEM((_N_NODES,), jnp.float32),
            pltpu.VMEM((2, _SZ), jnp.int32),
            pltpu.VMEM((_EPT,), jnp.float32),
            pltpu.SemaphoreType.DMA,
        ],
    )
    def sc_k(pq_hbm, ei_hbm, out_hbm, p_v, q_v, ei_v, o_v, sem):
        wid = lax.axis_index("s") * _NC + lax.axis_index("c")
        base = wid * _EPT
        base_al = (base // _ALN) * _ALN
        off0 = base - base_al
        c1 = pltpu.async_copy(pq_hbm.at[pl.ds(0, _N_NODES)], p_v, sem)
        c2 = pltpu.async_copy(pq_hbm.at[pl.ds(_N_NODES, _N_NODES)], q_v, sem)
        c3 = pltpu.async_copy(ei_hbm.at[:, pl.ds(base_al, _SZ)], ei_v, sem)
        c1.wait()
        c2.wait()
        c3.wait()

        @plsc.parallel_loop(0, _EPT, step=_L, unroll=8)
        def _(off):
            sidx = ei_v[0, pl.ds(off0 + off, _L)]
            didx = ei_v[1, pl.ds(off0 + off, _L)]
            pv = plsc.load_gather(p_v, [sidx])
            qv = plsc.load_gather(q_v, [didx])
            x = pv + qv
            o_v[pl.ds(off, _L)] = 1.0 / (1.0 + jnp.exp(-x))

        pltpu.sync_copy(o_v, out_hbm.at[pl.ds(base, _EPT)])

    return sc_k


_sc_kernel = _make_sc_kernel()


def kernel(h, edge_index, W, b):
    w2 = W.reshape(2, _D_FEAT)
    ei = edge_index.astype(jnp.int32)
    pq = pl.pallas_call(
        _tc_body,
        out_shape=jax.ShapeDtypeStruct((2 * _N_NODES,), jnp.float32),
        in_specs=[
            pl.BlockSpec(memory_space=pltpu.VMEM),
            pl.BlockSpec(memory_space=pltpu.VMEM),
            pl.BlockSpec(memory_space=pltpu.SMEM),
        ],
        out_specs=pl.BlockSpec(memory_space=pltpu.VMEM),
    )(h, w2, b)
    scores = _sc_kernel(pq, ei)
    out = scores.reshape(_N_EDGES, 1)
    return (out, out)
